# baseline (speedup 1.0000x reference)
import functools

import jax
import jax.numpy as jnp
from jax import lax
from jax.experimental import pallas as pl
from jax.experimental.pallas import tpu as pltpu
from jax.experimental.pallas import tpu_sc as plsc

N = 10000
E = 320000
D = 128
H = 128
NC = 2
NS = 16
NW = NC * NS
CH = 128
NCHUNK = 80
CHA = 64
NCHA = 160
EW = CH * NCHUNK
EPAD = NW * EW
NACC = 10240
RPT = NACC // NS
RCH = 128
BR = 2000

_mesh = plsc.VectorSubcoreMesh(core_axis_name="c", subcore_axis_name="s")


@functools.partial(
    pl.kernel,
    out_type=jax.ShapeDtypeStruct((NC, NACC, 16), jnp.float32),
    mesh=_mesh,
    scratch_types=[
        pltpu.VMEM((NCHUNK, CH), jnp.int32),
        pltpu.VMEM((CH, 16), jnp.float32),
        pltpu.VMEM_SHARED((NACC, 16), jnp.float32),
    ],
)
def _deg_kernel(dst_hbm, out_hbm, dst_v, buf_v, acc_sh):
    c = lax.axis_index("c")
    s = lax.axis_index("s")
    w = c * NS + s

    pltpu.sync_copy(dst_hbm.at[w], dst_v)

    def _zero(r, carry):
        buf_v[r, :] = jnp.zeros((16,), jnp.float32)
        return carry

    lax.fori_loop(0, CH, _zero, 0)
    for k in range(RPT // RCH):
        pltpu.sync_copy(buf_v.at[pl.ds(0, RCH)],
                        acc_sh.at[pl.ds(s * RPT + k * RCH, RCH)])
    plsc.subcore_barrier()

    def _ones(r, carry):
        buf_v[r, :] = jnp.ones((16,), jnp.float32)
        return carry

    lax.fori_loop(0, CH, _ones, 0)

    def _body(j, carry):
        pltpu.sync_copy(buf_v, acc_sh.at[dst_v.at[j]], add=True)
        return carry

    lax.fori_loop(0, NCHUNK, _body, 0)
    plsc.subcore_barrier()

    for k in range(RPT // RCH):
        pltpu.sync_copy(acc_sh.at[pl.ds(s * RPT + k * RCH, RCH)],
                        buf_v.at[pl.ds(0, RCH)])
        pltpu.sync_copy(buf_v.at[pl.ds(0, RCH)],
                        out_hbm.at[c, pl.ds(s * RPT + k * RCH, RCH)])


@functools.partial(
    pl.kernel,
    out_type=jax.ShapeDtypeStruct((NC, NACC, H), jnp.float32),
    mesh=_mesh,
    scratch_types=[
        pltpu.VMEM((NCHUNK, CH), jnp.int32),
        pltpu.VMEM((NCHUNK, CH), jnp.int32),
        pltpu.VMEM((CHA, H), jnp.float32),
        pltpu.VMEM((CHA, H), jnp.float32),
        pltpu.VMEM_SHARED((NACC, H), jnp.float32),
        pltpu.SemaphoreType.DMA,
        pltpu.SemaphoreType.DMA,
    ],
)
def _agg_kernel(src_hbm, dst_hbm, g_hbm, out_hbm, src_v, dst_v, rows0_v,
                rows1_v, acc_sh, sem0, sem1):
    c = lax.axis_index("c")
    s = lax.axis_index("s")
    w = c * NS + s

    pltpu.sync_copy(src_hbm.at[w], src_v)
    pltpu.sync_copy(dst_hbm.at[w], dst_v)

    def _zero(r, carry):
        for k in range(H // 16):
            rows0_v[r, pl.ds(k * 16, 16)] = jnp.zeros((16,), jnp.float32)
        return carry

    lax.fori_loop(0, CHA, _zero, 0)
    for k in range(RPT // CHA):
        pltpu.sync_copy(rows0_v,
                        acc_sh.at[pl.ds(s * RPT + k * CHA, CHA)])
    plsc.subcore_barrier()

    pltpu.sync_copy(g_hbm.at[src_v.at[0, pl.ds(0, CHA)]], rows0_v)

    def _body(t, carry):
        d0 = pltpu.async_copy(rows0_v, acc_sh.at[dst_v.at[t, pl.ds(0, CHA)]],
                              sem0, add=True)
        pltpu.sync_copy(g_hbm.at[src_v.at[t, pl.ds(CHA, CHA)]], rows1_v)
        d0.wait()
        d1 = pltpu.async_copy(rows1_v,
                              acc_sh.at[dst_v.at[t, pl.ds(CHA, CHA)]],
                              sem1, add=True)

        @pl.when(t + 1 < NCHA // 2)
        def _():
            pltpu.sync_copy(g_hbm.at[src_v.at[t + 1, pl.ds(0, CHA)]],
                            rows0_v)

        d1.wait()
        return carry

    lax.fori_loop(0, NCHA // 2, _body, 0)
    plsc.subcore_barrier()

    for k in range(RPT // CHA):
        pltpu.sync_copy(acc_sh.at[pl.ds(s * RPT + k * CHA, CHA)],
                        rows0_v)
        pltpu.sync_copy(rows0_v,
                        out_hbm.at[c, pl.ds(s * RPT + k * CHA, CHA)])


def _b1_body(deg_ref, x_ref, w_ref, g_ref, dinv_ref):
    dsum = deg_ref[0, :, 0:1] + deg_ref[1, :, 0:1]
    di = lax.rsqrt(1.0 + dsum)
    h = jnp.dot(x_ref[...], w_ref[...], preferred_element_type=jnp.float32)
    g_ref[...] = h * di
    dinv_ref[...] = di


def _b2_body(agg_ref, g_ref, dinv_ref, b_ref, a_ref, w_ref, g2_ref):
    di = dinv_ref[...]
    z = (agg_ref[0] + agg_ref[1] + g_ref[...]) * di + b_ref[...]
    z = jnp.where(z >= 0.0, z, z * a_ref[...])
    g2_ref[...] = jnp.dot(z, w_ref[...], preferred_element_type=jnp.float32) * di


def _b3_body(agg_ref, g_ref, dinv_ref, b_ref, a_ref, o_ref):
    di = dinv_ref[...]
    z = (agg_ref[0] + agg_ref[1] + g_ref[...]) * di + b_ref[...]
    o_ref[...] = jnp.where(z >= 0.0, z, z * a_ref[...])


def kernel(x, edge_index, W1, b1, W2, b2, alpha):
    src = edge_index[0]
    dst = edge_index[1]
    pad = EPAD - E
    srcp = jnp.concatenate(
        [src, jnp.zeros((pad,), jnp.int32)]).reshape(NW, NCHUNK, CH)
    dstp = jnp.concatenate(
        [dst, jnp.full((pad,), N, jnp.int32)]).reshape(NW, NCHUNK, CH)

    deg2 = _deg_kernel(dstp)

    g1, dinv = pl.pallas_call(
        _b1_body,
        grid=(N // BR,),
        in_specs=[
            pl.BlockSpec((2, BR, 16), lambda i: (0, i, 0)),
            pl.BlockSpec((BR, D), lambda i: (i, 0)),
            pl.BlockSpec((D, H), lambda i: (0, 0)),
        ],
        out_specs=[
            pl.BlockSpec((BR, H), lambda i: (i, 0)),
            pl.BlockSpec((BR, 1), lambda i: (i, 0)),
        ],
        out_shape=[
            jax.ShapeDtypeStruct((N, H), jnp.float32),
            jax.ShapeDtypeStruct((N, 1), jnp.float32),
        ],
    )(deg2, x, W1)

    agg1 = _agg_kernel(srcp, dstp, g1)

    g2 = pl.pallas_call(
        _b2_body,
        grid=(N // BR,),
        in_specs=[
            pl.BlockSpec((2, BR, H), lambda i: (0, i, 0)),
            pl.BlockSpec((BR, H), lambda i: (i, 0)),
            pl.BlockSpec((BR, 1), lambda i: (i, 0)),
            pl.BlockSpec((1, H), lambda i: (0, 0)),
            pl.BlockSpec((1, H), lambda i: (0, 0)),
            pl.BlockSpec((H, H), lambda i: (0, 0)),
        ],
        out_specs=pl.BlockSpec((BR, H), lambda i: (i, 0)),
        out_shape=jax.ShapeDtypeStruct((N, H), jnp.float32),
    )(agg1, g1, dinv, b1.reshape(1, H), alpha.reshape(1, H), W2)

    agg2 = _agg_kernel(srcp, dstp, g2)

    out = pl.pallas_call(
        _b3_body,
        grid=(N // BR,),
        in_specs=[
            pl.BlockSpec((2, BR, H), lambda i: (0, i, 0)),
            pl.BlockSpec((BR, H), lambda i: (i, 0)),
            pl.BlockSpec((BR, 1), lambda i: (i, 0)),
            pl.BlockSpec((1, H), lambda i: (0, 0)),
            pl.BlockSpec((1, H), lambda i: (0, 0)),
        ],
        out_specs=pl.BlockSpec((BR, H), lambda i: (i, 0)),
        out_shape=jax.ShapeDtypeStruct((N, H), jnp.float32),
    )(agg2, g2, dinv, b2.reshape(1, H), alpha.reshape(1, H))

    return out


# speedup vs baseline: 1.0101x; 1.0101x over previous
import functools

import jax
import jax.numpy as jnp
from jax import lax
from jax.experimental import pallas as pl
from jax.experimental.pallas import tpu as pltpu
from jax.experimental.pallas import tpu_sc as plsc

N = 10000
E = 320000
D = 128
H = 128
NC = 2
NS = 16
NW = NC * NS
CH = 128
NCHUNK = 80
CHA = 64
NCHA = 160
EW = CH * NCHUNK
EPAD = NW * EW
NACC = 10240
RPT = NACC // NS
RCH = 128
BR = 2000

_mesh = plsc.VectorSubcoreMesh(core_axis_name="c", subcore_axis_name="s")


@functools.partial(
    pl.kernel,
    out_type=jax.ShapeDtypeStruct((NC, NACC, 16), jnp.float32),
    mesh=_mesh,
    scratch_types=[
        pltpu.VMEM((NCHUNK, CH), jnp.int32),
        pltpu.VMEM((CH, 16), jnp.float32),
        pltpu.VMEM_SHARED((NACC, 16), jnp.float32),
    ],
)
def _deg_kernel(dst_hbm, out_hbm, dst_v, buf_v, acc_sh):
    c = lax.axis_index("c")
    s = lax.axis_index("s")
    w = c * NS + s

    pltpu.sync_copy(dst_hbm.at[w], dst_v)

    def _zero(r, carry):
        buf_v[r, :] = jnp.zeros((16,), jnp.float32)
        return carry

    lax.fori_loop(0, CH, _zero, 0)
    for k in range(RPT // RCH):
        pltpu.sync_copy(buf_v.at[pl.ds(0, RCH)],
                        acc_sh.at[pl.ds(s * RPT + k * RCH, RCH)])
    plsc.subcore_barrier()

    def _ones(r, carry):
        buf_v[r, :] = jnp.ones((16,), jnp.float32)
        return carry

    lax.fori_loop(0, CH, _ones, 0)

    def _body(j, carry):
        pltpu.sync_copy(buf_v, acc_sh.at[dst_v.at[j]], add=True)
        return carry

    lax.fori_loop(0, NCHUNK, _body, 0)
    plsc.subcore_barrier()

    for k in range(RPT // RCH):
        pltpu.sync_copy(acc_sh.at[pl.ds(s * RPT + k * RCH, RCH)],
                        buf_v.at[pl.ds(0, RCH)])
        pltpu.sync_copy(buf_v.at[pl.ds(0, RCH)],
                        out_hbm.at[c, pl.ds(s * RPT + k * RCH, RCH)])


@functools.partial(
    pl.kernel,
    out_type=jax.ShapeDtypeStruct((NC, NACC, H), jnp.float32),
    mesh=_mesh,
    scratch_types=[
        pltpu.VMEM((NCHUNK, CH), jnp.int32),
        pltpu.VMEM((NCHUNK, CH), jnp.int32),
        pltpu.VMEM((CHA, H), jnp.float32),
        pltpu.VMEM((CHA, H), jnp.float32),
        pltpu.VMEM_SHARED((NACC, H), jnp.float32),
        pltpu.SemaphoreType.DMA,
        pltpu.SemaphoreType.DMA,
        pltpu.SemaphoreType.DMA,
        pltpu.SemaphoreType.DMA,
    ],
)
def _agg_kernel(src_hbm, dst_hbm, g_hbm, out_hbm, src_v, dst_v, rows0_v,
                rows1_v, acc_sh, sem0, sem1, sem2, sem3):
    c = lax.axis_index("c")
    s = lax.axis_index("s")
    w = c * NS + s

    pltpu.sync_copy(src_hbm.at[w], src_v)
    pltpu.sync_copy(dst_hbm.at[w], dst_v)

    def _zero(r, carry):
        for k in range(H // 16):
            rows0_v[r, pl.ds(k * 16, 16)] = jnp.zeros((16,), jnp.float32)
        return carry

    lax.fori_loop(0, CHA, _zero, 0)
    for k in range(RPT // CHA):
        pltpu.sync_copy(rows0_v,
                        acc_sh.at[pl.ds(s * RPT + k * CHA, CHA)])
    plsc.subcore_barrier()

    def _body(t, carry):
        d0 = pltpu.async_copy(g_hbm.at[src_v.at[t, pl.ds(0, CHA)]],
                              rows0_v, sem0)
        d1 = pltpu.async_copy(g_hbm.at[src_v.at[t, pl.ds(CHA, CHA)]],
                              rows1_v, sem1)
        d0.wait()
        s0 = pltpu.async_copy(rows0_v,
                              acc_sh.at[dst_v.at[t, pl.ds(0, CHA)]],
                              sem2, add=True)
        d1.wait()
        s1 = pltpu.async_copy(rows1_v,
                              acc_sh.at[dst_v.at[t, pl.ds(CHA, CHA)]],
                              sem3, add=True)
        s0.wait()
        s1.wait()
        return carry

    lax.fori_loop(0, NCHA // 2, _body, 0)
    plsc.subcore_barrier()

    for k in range(RPT // CHA):
        pltpu.sync_copy(acc_sh.at[pl.ds(s * RPT + k * CHA, CHA)],
                        rows0_v)
        pltpu.sync_copy(rows0_v,
                        out_hbm.at[c, pl.ds(s * RPT + k * CHA, CHA)])


def _b1_body(deg_ref, x_ref, w_ref, g_ref, dinv_ref):
    dsum = deg_ref[0, :, 0:1] + deg_ref[1, :, 0:1]
    di = lax.rsqrt(1.0 + dsum)
    h = jnp.dot(x_ref[...], w_ref[...], preferred_element_type=jnp.float32)
    g_ref[...] = h * di
    dinv_ref[...] = di


def _b2_body(agg_ref, g_ref, dinv_ref, b_ref, a_ref, w_ref, g2_ref):
    di = dinv_ref[...]
    z = (agg_ref[0] + agg_ref[1] + g_ref[...]) * di + b_ref[...]
    z = jnp.where(z >= 0.0, z, z * a_ref[...])
    g2_ref[...] = jnp.dot(z, w_ref[...], preferred_element_type=jnp.float32) * di


def _b3_body(agg_ref, g_ref, dinv_ref, b_ref, a_ref, o_ref):
    di = dinv_ref[...]
    z = (agg_ref[0] + agg_ref[1] + g_ref[...]) * di + b_ref[...]
    o_ref[...] = jnp.where(z >= 0.0, z, z * a_ref[...])


def kernel(x, edge_index, W1, b1, W2, b2, alpha):
    src = edge_index[0]
    dst = edge_index[1]
    pad = EPAD - E
    srcp = jnp.concatenate(
        [src, jnp.zeros((pad,), jnp.int32)]).reshape(NW, NCHUNK, CH)
    dstp = jnp.concatenate(
        [dst, jnp.full((pad,), N, jnp.int32)]).reshape(NW, NCHUNK, CH)

    deg2 = _deg_kernel(dstp)

    g1, dinv = pl.pallas_call(
        _b1_body,
        grid=(N // BR,),
        in_specs=[
            pl.BlockSpec((2, BR, 16), lambda i: (0, i, 0)),
            pl.BlockSpec((BR, D), lambda i: (i, 0)),
            pl.BlockSpec((D, H), lambda i: (0, 0)),
        ],
        out_specs=[
            pl.BlockSpec((BR, H), lambda i: (i, 0)),
            pl.BlockSpec((BR, 1), lambda i: (i, 0)),
        ],
        out_shape=[
            jax.ShapeDtypeStruct((N, H), jnp.float32),
            jax.ShapeDtypeStruct((N, 1), jnp.float32),
        ],
    )(deg2, x, W1)

    agg1 = _agg_kernel(srcp, dstp, g1)

    g2 = pl.pallas_call(
        _b2_body,
        grid=(N // BR,),
        in_specs=[
            pl.BlockSpec((2, BR, H), lambda i: (0, i, 0)),
            pl.BlockSpec((BR, H), lambda i: (i, 0)),
            pl.BlockSpec((BR, 1), lambda i: (i, 0)),
            pl.BlockSpec((1, H), lambda i: (0, 0)),
            pl.BlockSpec((1, H), lambda i: (0, 0)),
            pl.BlockSpec((H, H), lambda i: (0, 0)),
        ],
        out_specs=pl.BlockSpec((BR, H), lambda i: (i, 0)),
        out_shape=jax.ShapeDtypeStruct((N, H), jnp.float32),
    )(agg1, g1, dinv, b1.reshape(1, H), alpha.reshape(1, H), W2)

    agg2 = _agg_kernel(srcp, dstp, g2)

    out = pl.pallas_call(
        _b3_body,
        grid=(N // BR,),
        in_specs=[
            pl.BlockSpec((2, BR, H), lambda i: (0, i, 0)),
            pl.BlockSpec((BR, H), lambda i: (i, 0)),
            pl.BlockSpec((BR, 1), lambda i: (i, 0)),
            pl.BlockSpec((1, H), lambda i: (0, 0)),
            pl.BlockSpec((1, H), lambda i: (0, 0)),
        ],
        out_specs=pl.BlockSpec((BR, H), lambda i: (i, 0)),
        out_shape=jax.ShapeDtypeStruct((N, H), jnp.float32),
    )(agg2, g2, dinv, b2.reshape(1, H), alpha.reshape(1, H))

    return out


# speedup vs baseline: 2.9373x; 2.9080x over previous
import functools

import jax
import jax.numpy as jnp
from jax import lax
from jax.experimental import pallas as pl
from jax.experimental.pallas import tpu as pltpu
from jax.experimental.pallas import tpu_sc as plsc

N = 10000
E = 320000
D = 128
H = 128
NC = 2
NS = 16
NW = NC * NS
CH = 128
NCHUNK = 80
CHA = 64
NCHA = 160
EW = CH * NCHUNK
EPAD = NW * EW
NACC = 10240
RPT = NACC // NS
RCH = 128
BR = 2000

_mesh = plsc.VectorSubcoreMesh(core_axis_name="c", subcore_axis_name="s")


@functools.partial(
    pl.kernel,
    out_type=jax.ShapeDtypeStruct((NC, NACC, 16), jnp.float32),
    mesh=_mesh,
    scratch_types=[
        pltpu.VMEM((NCHUNK, CH), jnp.int32),
        pltpu.VMEM((CH, 16), jnp.float32),
        pltpu.VMEM_SHARED((NACC, 16), jnp.float32),
    ],
)
def _deg_kernel(dst_hbm, out_hbm, dst_v, buf_v, acc_sh):
    c = lax.axis_index("c")
    s = lax.axis_index("s")
    w = c * NS + s

    pltpu.sync_copy(dst_hbm.at[w], dst_v)

    def _zero(r, carry):
        buf_v[r, :] = jnp.zeros((16,), jnp.float32)
        return carry

    lax.fori_loop(0, CH, _zero, 0)
    for k in range(RPT // RCH):
        pltpu.sync_copy(buf_v.at[pl.ds(0, RCH)],
                        acc_sh.at[pl.ds(s * RPT + k * RCH, RCH)])
    plsc.subcore_barrier()

    def _ones(r, carry):
        buf_v[r, :] = jnp.ones((16,), jnp.float32)
        return carry

    lax.fori_loop(0, CH, _ones, 0)

    def _body(j, carry):
        pltpu.sync_copy(buf_v, acc_sh.at[dst_v.at[j]], add=True)
        return carry

    lax.fori_loop(0, NCHUNK, _body, 0)
    plsc.subcore_barrier()

    for k in range(RPT // RCH):
        pltpu.sync_copy(acc_sh.at[pl.ds(s * RPT + k * RCH, RCH)],
                        buf_v.at[pl.ds(0, RCH)])
        pltpu.sync_copy(buf_v.at[pl.ds(0, RCH)],
                        out_hbm.at[c, pl.ds(s * RPT + k * RCH, RCH)])


@functools.partial(
    pl.kernel,
    out_type=jax.ShapeDtypeStruct((NC, NACC, H), jnp.float32),
    mesh=_mesh,
    scratch_types=[
        pltpu.VMEM((NCHUNK, CH), jnp.int32),
        pltpu.VMEM((NCHUNK, CH), jnp.int32),
        pltpu.VMEM((CHA, H), jnp.float32),
        pltpu.VMEM((CHA, H), jnp.float32),
        pltpu.VMEM_SHARED((NACC, H), jnp.float32),
        pltpu.SemaphoreType.DMA,
        pltpu.SemaphoreType.DMA,
        pltpu.SemaphoreType.DMA,
        pltpu.SemaphoreType.DMA,
    ],
)
def _agg_kernel(src_hbm, dst_hbm, g_hbm, out_hbm, src_v, dst_v, rows0_v,
                rows1_v, acc_sh, sem0, sem1, sem2, sem3):
    c = lax.axis_index("c")
    s = lax.axis_index("s")
    w = c * NS + s

    pltpu.sync_copy(src_hbm.at[w], src_v)
    pltpu.sync_copy(dst_hbm.at[w], dst_v)

    def _zero(r, carry):
        for k in range(H // 16):
            rows0_v[r, pl.ds(k * 16, 16)] = jnp.zeros((16,), jnp.float32)
        return carry

    lax.fori_loop(0, CHA, _zero, 0)
    for k in range(RPT // CHA):
        pltpu.sync_copy(rows0_v,
                        acc_sh.at[pl.ds(s * RPT + k * CHA, CHA)])
    plsc.subcore_barrier()

    def _body(t, carry):
        d0 = pltpu.async_copy(g_hbm.at[src_v.at[t, pl.ds(0, CHA)]],
                              rows0_v, sem0)
        d1 = pltpu.async_copy(g_hbm.at[src_v.at[t, pl.ds(CHA, CHA)]],
                              rows1_v, sem1)
        d0.wait()
        s0 = pltpu.async_copy(rows0_v,
                              acc_sh.at[dst_v.at[t, pl.ds(0, CHA)]],
                              sem2, add=True)
        d1.wait()
        s1 = pltpu.async_copy(rows1_v,
                              acc_sh.at[dst_v.at[t, pl.ds(CHA, CHA)]],
                              sem3, add=True)
        s0.wait()
        s1.wait()
        return carry

    lax.fori_loop(0, NCHA // 2, _body, 0)
    plsc.subcore_barrier()

    for k in range(RPT // CHA):
        pltpu.sync_copy(acc_sh.at[pl.ds(s * RPT + k * CHA, CHA)],
                        rows0_v)
        pltpu.sync_copy(rows0_v,
                        out_hbm.at[c, pl.ds(s * RPT + k * CHA, CHA)])


def _b1_body(deg_ref, x_ref, w_ref, g_ref, dinv_ref):
    dsum = deg_ref[0, :, 0:1] + deg_ref[1, :, 0:1]
    di = lax.rsqrt(1.0 + dsum)
    h = jnp.dot(x_ref[...], w_ref[...], preferred_element_type=jnp.float32)
    g_ref[...] = h * di
    dinv_ref[...] = di


def _b2_body(agg_ref, g_ref, dinv_ref, b_ref, a_ref, w_ref, g2_ref):
    di = dinv_ref[...]
    z = (agg_ref[0] + agg_ref[1] + g_ref[...]) * di + b_ref[...]
    z = jnp.where(z >= 0.0, z, z * a_ref[...])
    g2_ref[...] = jnp.dot(z, w_ref[...], preferred_element_type=jnp.float32) * di


def _b3_body(agg_ref, g_ref, dinv_ref, b_ref, a_ref, o_ref):
    di = dinv_ref[...]
    z = (agg_ref[0] + agg_ref[1] + g_ref[...]) * di + b_ref[...]
    o_ref[...] = jnp.where(z >= 0.0, z, z * a_ref[...])


def kernel(x, edge_index, W1, b1, W2, b2, alpha):
    src = edge_index[0]
    dst = edge_index[1]
    pad = EPAD - E
    padi = jnp.arange(pad, dtype=jnp.int32)
    srcp = jnp.concatenate(
        [src, padi % N]).reshape(NW, NCHUNK, CH)
    dstp = jnp.concatenate(
        [dst, N + padi % (NACC - N)]).reshape(NW, NCHUNK, CH)

    deg2 = _deg_kernel(dstp)

    g1, dinv = pl.pallas_call(
        _b1_body,
        grid=(N // BR,),
        in_specs=[
            pl.BlockSpec((2, BR, 16), lambda i: (0, i, 0)),
            pl.BlockSpec((BR, D), lambda i: (i, 0)),
            pl.BlockSpec((D, H), lambda i: (0, 0)),
        ],
        out_specs=[
            pl.BlockSpec((BR, H), lambda i: (i, 0)),
            pl.BlockSpec((BR, 1), lambda i: (i, 0)),
        ],
        out_shape=[
            jax.ShapeDtypeStruct((N, H), jnp.float32),
            jax.ShapeDtypeStruct((N, 1), jnp.float32),
        ],
    )(deg2, x, W1)

    agg1 = _agg_kernel(srcp, dstp, g1)

    g2 = pl.pallas_call(
        _b2_body,
        grid=(N // BR,),
        in_specs=[
            pl.BlockSpec((2, BR, H), lambda i: (0, i, 0)),
            pl.BlockSpec((BR, H), lambda i: (i, 0)),
            pl.BlockSpec((BR, 1), lambda i: (i, 0)),
            pl.BlockSpec((1, H), lambda i: (0, 0)),
            pl.BlockSpec((1, H), lambda i: (0, 0)),
            pl.BlockSpec((H, H), lambda i: (0, 0)),
        ],
        out_specs=pl.BlockSpec((BR, H), lambda i: (i, 0)),
        out_shape=jax.ShapeDtypeStruct((N, H), jnp.float32),
    )(agg1, g1, dinv, b1.reshape(1, H), alpha.reshape(1, H), W2)

    agg2 = _agg_kernel(srcp, dstp, g2)

    out = pl.pallas_call(
        _b3_body,
        grid=(N // BR,),
        in_specs=[
            pl.BlockSpec((2, BR, H), lambda i: (0, i, 0)),
            pl.BlockSpec((BR, H), lambda i: (i, 0)),
            pl.BlockSpec((BR, 1), lambda i: (i, 0)),
            pl.BlockSpec((1, H), lambda i: (0, 0)),
            pl.BlockSpec((1, H), lambda i: (0, 0)),
        ],
        out_specs=pl.BlockSpec((BR, H), lambda i: (i, 0)),
        out_shape=jax.ShapeDtypeStruct((N, H), jnp.float32),
    )(agg2, g2, dinv, b2.reshape(1, H), alpha.reshape(1, H))

    return out
